# Initial kernel scaffold; baseline (speedup 1.0000x reference)
import jax
import jax.numpy as jnp
from jax.experimental import pallas as pl


def kernel(feat, edge_index, edge_weight, W, prelu_a):
    raise NotImplementedError("write your pallas kernel here")


# speedup vs baseline: 2.3934x; 2.3934x over previous
import functools

import jax
import jax.numpy as jnp
from jax import lax
from jax.experimental import pallas as pl
from jax.experimental.pallas import tpu as pltpu
from jax.experimental.pallas import tpu_sc as plsc

N_NODES = 10000
N_PAD = 10240
E = 160000
E_PAD = 163840
F = 256
H = 128
NC = 2
NS = 16
C = 128
EDGES_PER_SUB = E_PAD // NS
NCHUNK = EDGES_PER_SUB // C
ROWS_PER_SUB = N_PAD // NS
CB = 128


def _sc_body(feat2, rows2, colp, ewp, g_out, rowv, colv, wsm, rows, acc):
    c = lax.axis_index("c")
    s = lax.axis_index("s")

    def zrow(i, carry):
        for j in range(H // 16):
            rows[i, pl.ds(j * 16, 16)] = jnp.zeros((16,), jnp.float32)
        return carry

    lax.fori_loop(0, CB, zrow, 0)
    stripe = s * ROWS_PER_SUB
    for i in range(ROWS_PER_SUB // CB):
        pltpu.sync_copy(rows, acc.at[pl.ds(stripe + i * CB, CB)])
    plsc.subcore_barrier()

    ebase = s * EDGES_PER_SUB

    def chunk(k, carry):
        base = ebase + k * C
        pltpu.sync_copy(rows2.at[c, pl.ds(base, C)], rowv)
        pltpu.sync_copy(colp.at[pl.ds(base, C)], colv)
        pltpu.sync_copy(ewp.at[pl.ds(base, C)], wsm)
        pltpu.sync_copy(feat2.at[rowv], rows)

        def emul(gidx, carry2):
            wvec = wsm[pl.ds(gidx * 16, 16)]
            for lane in range(16):
                bidx = jnp.full((16,), lane, jnp.int32)
                wb = wvec.at[bidx].get(mode="promise_in_bounds")
                e = gidx * 16 + lane
                for j in range(H // 16):
                    rows[e, pl.ds(j * 16, 16)] = rows[e, pl.ds(j * 16, 16)] * wb
            return carry2

        lax.fori_loop(0, C // 16, emul, 0)
        pltpu.sync_copy(rows, acc.at[colv], add=True)
        return carry

    lax.fori_loop(0, NCHUNK, chunk, 0)
    plsc.subcore_barrier()

    for i in range(ROWS_PER_SUB // CB):
        r0 = stripe + i * CB
        pltpu.sync_copy(acc.at[pl.ds(r0, CB)], rows)
        pltpu.sync_copy(rows, g_out.at[pl.ds(r0, CB), pl.ds(c * H, H)])


_sc_scatter = functools.partial(
    pl.kernel,
    mesh=plsc.VectorSubcoreMesh(core_axis_name="c", subcore_axis_name="s"),
    out_type=jax.ShapeDtypeStruct((N_PAD, F), jnp.float32),
    scratch_types=[
        pltpu.VMEM((C,), jnp.int32),
        pltpu.VMEM((C,), jnp.int32),
        pltpu.VMEM((C,), jnp.float32),
        pltpu.VMEM((CB, H), jnp.float32),
        pltpu.VMEM_SHARED((N_PAD, H), jnp.float32),
    ],
)(_sc_body)


def _mm_body(g_ref, wt_ref, a_ref, o_ref):
    x = jnp.dot(g_ref[...], wt_ref[...], preferred_element_type=jnp.float32)
    a = a_ref[0]
    o_ref[...] = jnp.where(x >= 0.0, x, a * x)


def _matmul_prelu(g, wt, a_arr):
    return pl.pallas_call(
        _mm_body,
        grid=(N_PAD // 1024,),
        in_specs=[
            pl.BlockSpec((1024, F), lambda i: (i, 0)),
            pl.BlockSpec((F, F), lambda i: (0, 0)),
            pl.BlockSpec(memory_space=pltpu.SMEM),
        ],
        out_specs=pl.BlockSpec((1024, F), lambda i: (i, 0)),
        out_shape=jax.ShapeDtypeStruct((N_PAD, F), jnp.float32),
    )(g, wt, a_arr)


@jax.jit
def kernel(feat, edge_index, edge_weight, W, prelu_a):
    row = edge_index[0].astype(jnp.int32)
    col = edge_index[1].astype(jnp.int32)
    zpad = jnp.zeros((E_PAD - E,), jnp.int32)
    rowp = jnp.concatenate([row, zpad])
    colp = jnp.concatenate([col, zpad])
    ewp = jnp.concatenate([edge_weight, jnp.zeros((E_PAD - E,), jnp.float32)])
    rows2 = jnp.stack([rowp, rowp + N_NODES])
    feat2 = jnp.concatenate([feat[:, :H], feat[:, H:]], axis=0)

    g = _sc_scatter(feat2, rows2, colp, ewp)

    out = _matmul_prelu(g, W.T, prelu_a.reshape(1))
    return out[:N_NODES]


# speedup vs baseline: 3.1406x; 1.3122x over previous
import functools

import jax
import jax.numpy as jnp
from jax import lax
from jax.experimental import pallas as pl
from jax.experimental.pallas import tpu as pltpu
from jax.experimental.pallas import tpu_sc as plsc

N_NODES = 10000
N_PAD = 10240
E = 160000
E_PAD = 163840
F = 256
H = 128
NC = 2
NS = 16
C = 128
EDGES_PER_SUB = E_PAD // NS
NCHUNK = EDGES_PER_SUB // C
ROWS_PER_SUB = N_PAD // NS
CB = 128


def _sc_body(feat2, meta, ewp, g_out, mbuf, wv0, wv1, rows, acc, semg, semi, semw):
    c = lax.axis_index("c")
    s = lax.axis_index("s")

    def zrow(i, carry):
        for t in range(H // 16):
            rows[0, i, pl.ds(t * 16, 16)] = jnp.zeros((16,), jnp.float32)
        return carry

    lax.fori_loop(0, CB, zrow, 0)
    stripe = s * ROWS_PER_SUB
    for i in range(ROWS_PER_SUB // CB):
        pltpu.sync_copy(rows.at[0], acc.at[pl.ds(stripe + i * CB, CB)])
    plsc.subcore_barrier()

    cbase = s * NCHUNK

    def meta_start(j, b, wvb):
        pltpu.async_copy(meta.at[c, cbase + j], mbuf.at[b], semi.at[b])
        pltpu.async_copy(ewp.at[pl.ds((cbase + j) * C, C)], wvb, semw.at[b])

    def meta_wait(j, b, wvb):
        pltpu.make_async_copy(meta.at[c, cbase + j], mbuf.at[b], semi.at[b]).wait()
        pltpu.make_async_copy(ewp.at[pl.ds((cbase + j) * C, C)], wvb, semw.at[b]).wait()

    def gather_start(b):
        pltpu.async_copy(feat2.at[mbuf.at[b, 0]], rows.at[b], semg.at[b])

    def gather_wait(b):
        pltpu.make_async_copy(
            feat2.at[mbuf.at[b, 0]], rows.at[b], semg.at[b]
        ).wait()

    meta_start(0, 0, wv0)
    meta_start(1, 1, wv1)
    meta_wait(0, 0, wv0)
    gather_start(0)

    def do_slot(j, b, wvb, pf_gather, pf_meta):
        gather_wait(b)

        def emul(g, carry2):
            wvec = wvb[pl.ds(g * 16, 16)]
            for lane in range(16):
                bidx = jnp.full((16,), lane, jnp.int32)
                wb = wvec.at[bidx].get(mode="promise_in_bounds")
                e = g * 16 + lane
                for t in range(H // 16):
                    rows[b, e, pl.ds(t * 16, 16)] = (
                        rows[b, e, pl.ds(t * 16, 16)] * wb
                    )
            return carry2

        lax.fori_loop(0, C // 16, emul, 0)

        if pf_gather:
            nb = 1 - b
            nwvb = wv1 if b == 0 else wv0
            meta_wait(j + 1, nb, nwvb)
            gather_start(nb)
        pltpu.sync_copy(rows.at[b], acc.at[mbuf.at[b, 1]], add=True)
        if pf_meta:
            meta_start(j + 2, b, wvb)

    def pair(p, carry):
        j0 = p * 2
        do_slot(j0, 0, wv0, True, True)
        do_slot(j0 + 1, 1, wv1, True, True)
        return carry

    lax.fori_loop(0, NCHUNK // 2 - 1, pair, 0)
    do_slot(NCHUNK - 2, 0, wv0, True, False)
    do_slot(NCHUNK - 1, 1, wv1, False, False)
    plsc.subcore_barrier()

    for i in range(ROWS_PER_SUB // CB):
        r0 = stripe + i * CB
        pltpu.sync_copy(
            acc.at[pl.ds(r0, CB)], g_out.at[pl.ds(r0, CB), pl.ds(c * H, H)]
        )


_sc_scatter = functools.partial(
    pl.kernel,
    mesh=plsc.VectorSubcoreMesh(core_axis_name="c", subcore_axis_name="s"),
    out_type=jax.ShapeDtypeStruct((N_PAD, F), jnp.float32),
    scratch_types=[
        pltpu.VMEM((2, 2, C), jnp.int32),
        pltpu.VMEM((C,), jnp.float32),
        pltpu.VMEM((C,), jnp.float32),
        pltpu.VMEM((2, CB, H), jnp.float32),
        pltpu.VMEM_SHARED((N_PAD, H), jnp.float32),
        pltpu.SemaphoreType.DMA((2,)),
        pltpu.SemaphoreType.DMA((2,)),
        pltpu.SemaphoreType.DMA((2,)),
    ],
)(_sc_body)


def _mm_body(g_ref, wt_ref, a_ref, o_ref):
    x = jnp.dot(g_ref[...], wt_ref[...], preferred_element_type=jnp.float32)
    a = a_ref[0]
    o_ref[...] = jnp.where(x >= 0.0, x, a * x)


def _matmul_prelu(g, wt, a_arr):
    return pl.pallas_call(
        _mm_body,
        grid=(N_PAD // 1024,),
        in_specs=[
            pl.BlockSpec((1024, F), lambda i: (i, 0)),
            pl.BlockSpec((F, F), lambda i: (0, 0)),
            pl.BlockSpec(memory_space=pltpu.SMEM),
        ],
        out_specs=pl.BlockSpec((1024, F), lambda i: (i, 0)),
        out_shape=jax.ShapeDtypeStruct((N_PAD, F), jnp.float32),
    )(g, wt, a_arr)


@jax.jit
def kernel(feat, edge_index, edge_weight, W, prelu_a):
    row = edge_index[0].astype(jnp.int32)
    col = edge_index[1].astype(jnp.int32)
    zpad = jnp.zeros((E_PAD - E,), jnp.int32)
    rowp = jnp.concatenate([row, zpad])
    colp = jnp.concatenate([col, zpad])
    ewp = jnp.concatenate([edge_weight, jnp.zeros((E_PAD - E,), jnp.float32)])
    meta = jnp.stack(
        [
            jnp.stack([rowp, colp]),
            jnp.stack([rowp + N_NODES, colp]),
        ]
    )
    meta = meta.reshape(NC, 2, NS * NCHUNK, C).transpose(0, 2, 1, 3)
    feat2 = jnp.concatenate([feat[:, :H], feat[:, H:]], axis=0)

    g = _sc_scatter(feat2, meta, ewp)

    out = _matmul_prelu(g, W.T, prelu_a.reshape(1))
    return out[:N_NODES]
